# baseline (speedup 1.0000x reference)
import functools

import jax
import jax.numpy as jnp
from jax import lax
from jax.experimental import pallas as pl
from jax.experimental.pallas import tpu as pltpu
from jax.experimental.pallas import tpu_sc as plsc

NE = 100000
NR = 500
SD = 96
TD = 32
RD = SD + TD
B = 4096

NC = 2
NS = 16
NW = NC * NS
BPW = B // NW

_OUT_TYPE = (
    [jax.ShapeDtypeStruct((B, SD), jnp.float32)] * 4
    + [jax.ShapeDtypeStruct((B, RD), jnp.float32)] * 2
    + [jax.ShapeDtypeStruct((B, TD), jnp.float32)] * 36
)


@functools.cache
def _build_sc_gather():
  mesh = plsc.VectorSubcoreMesh(core_axis_name="c", subcore_axis_name="s")
  return functools.partial(
      pl.kernel,
      out_type=_OUT_TYPE,
      mesh=mesh,
      compiler_params=pltpu.CompilerParams(use_tc_tiling_on_sc=False),
      scratch_types=[
        pltpu.VMEM((BPW,), jnp.int32),
        pltpu.VMEM((BPW,), jnp.int32),
        pltpu.VMEM((BPW,), jnp.int32),
        pltpu.VMEM((BPW, SD), jnp.float32),
        pltpu.VMEM((BPW, SD), jnp.float32),
        pltpu.VMEM((BPW, RD), jnp.float32),
        pltpu.VMEM((BPW, RD), jnp.float32),
        pltpu.VMEM((BPW, TD), jnp.float32),
        pltpu.VMEM((BPW, TD), jnp.float32),
        pltpu.SemaphoreType.DMA,
        pltpu.SemaphoreType.DMA,
        pltpu.SemaphoreType.DMA,
        pltpu.SemaphoreType.DMA,
      ],
  )(_sc_gather_body)


def _sc_gather_body(s_hbm, o_hbm, r_hbm, es_hbm, eo_hbm, rf_hbm, ri_hbm, *rest):
    temp_hbm = rest[:18]
    outs = rest[18:60]
    (idx_s, idx_o, idx_r,
     be0, be1, br0, br1, bt0, bt1,
     g0, g1, w0, w1) = rest[60:]

    wid = lax.axis_index("s") * NC + lax.axis_index("c")
    base = wid * BPW

    pltpu.sync_copy(s_hbm.at[pl.ds(base, BPW)], idx_s)
    pltpu.sync_copy(o_hbm.at[pl.ds(base, BPW)], idx_o)
    pltpu.sync_copy(r_hbm.at[pl.ds(base, BPW)], idx_r)

    gsems = (g0, g1)
    wsems = (w0, w1)

    def run(jobs, bufs):
        n = len(jobs)
        gops = [None, None]
        wops = [None, None]
        gout = [None, None]
        for t in range(n + 1):
            p = t & 1
            if t < n:
                tab, idx, out = jobs[t]
                if wops[p] is not None:
                    wops[p].wait()
                    wops[p] = None
                gops[p] = pltpu.async_copy(tab.at[idx], bufs[p], gsems[p])
                gout[p] = out
            q = (t - 1) & 1
            if t >= 1 and gops[q] is not None:
                gops[q].wait()
                wops[q] = pltpu.async_copy(
                    bufs[q], gout[q].at[pl.ds(base, BPW)], wsems[q])
                gops[q] = None
        for p in (0, 1):
            if wops[p] is not None:
                wops[p].wait()

    run([(es_hbm, idx_s, outs[0]), (eo_hbm, idx_o, outs[1]),
         (es_hbm, idx_o, outs[2]), (eo_hbm, idx_s, outs[3])], (be0, be1))
    run([(rf_hbm, idx_r, outs[4]), (ri_hbm, idx_r, outs[5])], (br0, br1))
    tjobs = ([(temp_hbm[k], idx_s, outs[6 + k]) for k in range(18)]
             + [(temp_hbm[k], idx_o, outs[24 + k]) for k in range(18)])
    run(tjobs, (bt0, bt1))


_TB = 512


def _tc_body(*refs):
    y_ref, m_ref, d_ref, a1, a2, a3, a4, rf, ri = refs[:9]
    t = refs[9:45]
    out_ref = refs[45]

    yv = y_ref[...]
    mv = m_ref[...]
    dv = d_ref[...]

    def temb(p9):
        yf, yp, ya, mf, mp, ma, df, dp, da = p9
        return (ya[...] * jnp.sin(yf[...] * yv + yp[...])
                + ma[...] * jnp.sin(mf[...] * mv + mp[...])
                + da[...] * jnp.sin(df[...] * dv + dp[...]))

    t_ss = temb(t[0:9])
    t_so = temb(t[9:18])
    t_os = temb(t[18:27])
    t_oo = temb(t[27:36])

    rfv = rf[...]
    riv = ri[...]
    ent = a1[...] * rfv[:, :SD] * a2[...] + a3[...] * riv[:, :SD] * a4[...]
    tmp = t_ss * rfv[:, SD:] * t_oo + t_os * riv[:, SD:] * t_so
    out_ref[...] = 0.5 * (jnp.sum(ent, axis=1) + jnp.sum(tmp, axis=1))


def _tc_compute(y, m, d, gathered):
    grid = (B // _TB,)
    im = lambda i: (i, 0)
    in_specs = (
        [pl.BlockSpec((_TB, 1), im)] * 3
        + [pl.BlockSpec((_TB, SD), im)] * 4
        + [pl.BlockSpec((_TB, RD), im)] * 2
        + [pl.BlockSpec((_TB, TD), im)] * 36
    )
    return pl.pallas_call(
        _tc_body,
        grid=grid,
        in_specs=in_specs,
        out_specs=pl.BlockSpec((_TB,), lambda i: (i,)),
        out_shape=jax.ShapeDtypeStruct((B,), jnp.float32),
    )(y.reshape(B, 1), m.reshape(B, 1), d.reshape(B, 1), *gathered)


def kernel(s, r, o, y, m, d, s_t, s_e, o_t, o_e, e_emb_s, e_emb_o,
           r_emb_f, r_emb_i,
           y_frq_s, y_phi_s, y_amp_s, m_frq_s, m_phi_s, m_amp_s,
           d_frq_s, d_phi_s, d_amp_s,
           y_frq_o, y_phi_o, y_amp_o, m_frq_o, m_phi_o, m_amp_o,
           d_frq_o, d_phi_o, d_amp_o):
    temps = (y_frq_s, y_phi_s, y_amp_s, m_frq_s, m_phi_s, m_amp_s,
             d_frq_s, d_phi_s, d_amp_s,
             y_frq_o, y_phi_o, y_amp_o, m_frq_o, m_phi_o, m_amp_o,
             d_frq_o, d_phi_o, d_amp_o)
    gathered = _build_sc_gather()(
        s.astype(jnp.int32), o.astype(jnp.int32), r.astype(jnp.int32),
        e_emb_s, e_emb_o, r_emb_f, r_emb_i, *temps)
    return _tc_compute(y, m, d, gathered)
